# Initial kernel scaffold; baseline (speedup 1.0000x reference)
import jax
import jax.numpy as jnp
from jax.experimental import pallas as pl


def kernel(pos, flat_netpin, netpin_start, net_weights, net_mask, sll_counts_table):
    raise NotImplementedError("write your pallas kernel here")


# speedup vs baseline: 361.7246x; 361.7246x over previous
import functools

import jax
import jax.numpy as jnp
from jax import lax
from jax.experimental import pallas as pl
from jax.experimental.pallas import tpu as pltpu
from jax.experimental.pallas import tpu_sc as plsc

SLR_INV_H = 4.0
NUM_SLRY = 4
NC, NS, L = 2, 16, 16
NW = NC * NS
CH = 4096


def _ceil_to(x, m):
  return (x + m - 1) // m * m


@functools.partial(jax.jit, static_argnums=())
def kernel(pos, flat_netpin, netpin_start, net_weights, net_mask,
           sll_counts_table):
  P = flat_netpin.shape[0]
  N = netpin_start.shape[0] - 1
  NB = -(-N // NW)
  ROW = _ceil_to(NB + 1, 16)
  NBR = _ceil_to(NB, 16)
  Np = NW * NB

  pos_y = pos[P:]
  fnp_pad = jnp.concatenate(
      [flat_netpin, jnp.zeros((CH,), jnp.int32)])
  starts_ext = jnp.concatenate(
      [netpin_start, jnp.full((Np - N,), P, jnp.int32)])
  ridx = (jnp.arange(NW, dtype=jnp.int32)[:, None] * NB
          + jnp.arange(ROW, dtype=jnp.int32)[None, :])
  starts2d = jnp.where(ridx <= Np, starts_ext[jnp.minimum(ridx, Np)],
                       jnp.int32(P))
  w_ext = jnp.concatenate(
      [net_weights, jnp.zeros((Np - N,), jnp.float32)]).reshape(NW, NB)
  w2d = jnp.pad(w_ext, ((0, 0), (0, NBR - NB)))
  m_ext = jnp.concatenate(
      [net_mask.astype(jnp.float32),
       jnp.zeros((Np - N,), jnp.float32)]).reshape(NW, NB)
  m2d = jnp.pad(m_ext, ((0, 0), (0, NBR - NB)))
  tab_f = sll_counts_table.astype(jnp.float32)

  p0s = starts_ext[jnp.arange(NW, dtype=jnp.int32) * NB]
  p1s = starts_ext[(jnp.arange(NW, dtype=jnp.int32) + 1) * NB]
  a0s = (p0s // 8) * 8
  nchs = (p1s - a0s + CH - 1) // CH
  bounds2d = jnp.stack(
      [p0s, p1s, a0s, nchs] + [jnp.zeros((NW,), jnp.int32)] * 12,
      axis=1)

  steps = []
  s = 1
  while s <= NB:
    steps.append(s)
    s *= 2
  steps = tuple(reversed(steps))

  mesh = plsc.VectorSubcoreMesh(
      core_axis_name="c", subcore_axis_name="s",
      num_cores=NC, num_subcores=NS)

  @functools.partial(
      pl.kernel,
      out_type=jax.ShapeDtypeStruct((NW, L), jnp.float32),
      mesh=mesh,
      compiler_params=pltpu.CompilerParams(needs_layout_passes=False),
      scratch_types=dict(
          s_ref=pltpu.VMEM((ROW,), jnp.int32),
          occ_ref=pltpu.VMEM((NBR,), jnp.int32),
          w_ref=pltpu.VMEM((NBR,), jnp.float32),
          m_ref=pltpu.VMEM((NBR,), jnp.float32),
          idx_ref=pltpu.VMEM((CH,), jnp.int32),
          py_ref=pltpu.VMEM((CH,), jnp.float32),
          b_ref=pltpu.VMEM((16,), jnp.int32),
          tab_ref=pltpu.VMEM((16,), jnp.float32),
          sh_ref=pltpu.VMEM((48,), jnp.int32),
          bh_ref=pltpu.VMEM((48,), jnp.int32),
          outv_ref=pltpu.VMEM((L,), jnp.float32),
          sem=pltpu.SemaphoreType.DMA,
      ),
  )
  def sll_kernel(posy_hbm, fnp_hbm, starts_hbm, w_hbm, m_hbm, bnd_hbm,
                 tab_hbm, out_hbm, *, s_ref, occ_ref, w_ref, m_ref,
                 idx_ref, py_ref, b_ref, tab_ref, sh_ref, bh_ref,
                 outv_ref, sem):
    wid = lax.axis_index("s") * NC + lax.axis_index("c")
    lanes = lax.iota(jnp.int32, 16)

    pltpu.sync_copy(starts_hbm.at[wid], s_ref)
    pltpu.sync_copy(w_hbm.at[wid], w_ref)
    pltpu.sync_copy(m_hbm.at[wid], m_ref)
    pltpu.sync_copy(bnd_hbm.at[wid], b_ref)
    pltpu.sync_copy(tab_hbm, tab_ref)

    bv = b_ref[...]
    p0 = bv[0]
    p1 = bv[1]
    a0 = bv[2]
    nch = bv[3]

    sh_ref[pl.ds(0, 16)] = jnp.full((16,), -1, jnp.int32)
    sh_ref[pl.ds(32, 16)] = jnp.full((16,), -1, jnp.int32)
    bh_ref[pl.ds(0, 16)] = jnp.zeros((16,), jnp.int32)
    bh_ref[pl.ds(32, 16)] = jnp.zeros((16,), jnp.int32)

    def zbody(i, _):
      occ_ref[pl.ds(i * 16, 16)] = jnp.zeros((16,), jnp.int32)
      return 0

    lax.fori_loop(0, NBR // 16, zbody, 0)

    ones16 = jnp.full((16,), 1, jnp.int32)

    def chunk_body(c, _):
      base = pl.multiple_of(a0 + c * CH, 8)
      pltpu.sync_copy(fnp_hbm.at[pl.ds(base, CH)], idx_ref)
      pltpu.async_copy(posy_hbm.at[idx_ref], py_ref, sem).wait()

      def vbody(j, _):
        p = base + j * 16 + lanes
        py = py_ref[pl.ds(j * 16, 16)]
        sy = jnp.clip((py * SLR_INV_H).astype(jnp.int32), 0, NUM_SLRY - 1)
        valid = (p >= p0) & (p < p1)
        bit = jnp.where(valid, jnp.left_shift(ones16, sy),
                        jnp.zeros((16,), jnp.int32))
        k = jnp.zeros((16,), jnp.int32)
        for st in steps:
          cand = k + st
          sv = plsc.load_gather(s_ref, [jnp.minimum(cand, NB)])
          ok = (cand <= NB) & (sv <= p)
          k = jnp.where(ok, cand, k)
        seg = jnp.minimum(k, NB - 1)
        sh_ref[pl.ds(16, 16)] = seg
        bh_ref[pl.ds(16, 16)] = bit
        bacc = bit
        for d in (1, 2, 4, 8):
          sseg = sh_ref[pl.ds(16 - d, 16)]
          sbit = bh_ref[pl.ds(16 - d, 16)]
          bacc = bacc | jnp.where(sseg == seg, sbit,
                                  jnp.zeros((16,), jnp.int32))
          if d != 8:
            bh_ref[pl.ds(16, 16)] = bacc
        nxt = sh_ref[pl.ds(17, 16)]
        last = seg != nxt
        old = plsc.load_gather(occ_ref, [seg])
        plsc.store_scatter(occ_ref, [seg], old | bacc, mask=last)
        return 0

      lax.fori_loop(0, CH // 16, vbody, 0)
      return 0

    lax.fori_loop(0, nch, chunk_body, 0)

    def fbody(i, acc):
      occv = occ_ref[pl.ds(i * 16, 16)]
      sll = plsc.load_gather(tab_ref, [occv])
      w = w_ref[pl.ds(i * 16, 16)]
      m = m_ref[pl.ds(i * 16, 16)]
      return acc + w * m * sll

    acc = lax.fori_loop(0, NBR // 16, fbody,
                        jnp.zeros((L,), jnp.float32))
    outv_ref[...] = acc
    pltpu.sync_copy(outv_ref, out_hbm.at[wid])

  partials = sll_kernel(pos_y, fnp_pad, starts2d, w2d, m2d, bounds2d,
                        tab_f)
  return jnp.sum(partials)


# speedup vs baseline: 375.5997x; 1.0384x over previous
import functools

import jax
import jax.numpy as jnp
from jax import lax
from jax.experimental import pallas as pl
from jax.experimental.pallas import tpu as pltpu
from jax.experimental.pallas import tpu_sc as plsc

SLR_INV_H = 4.0
NUM_SLRY = 4
NC, NS, L = 2, 16, 16
NW = NC * NS
CH = 4096


def _ceil_to(x, m):
  return (x + m - 1) // m * m


@functools.partial(jax.jit, static_argnums=())
def kernel(pos, flat_netpin, netpin_start, net_weights, net_mask,
           sll_counts_table):
  P = flat_netpin.shape[0]
  N = netpin_start.shape[0] - 1
  NB = -(-N // NW)
  ROW = _ceil_to(NB + 1, 16)
  NBR = _ceil_to(NB, 16)
  Np = NW * NB

  pos_y = pos[P:]
  fnp_pad = jnp.concatenate(
      [flat_netpin, jnp.zeros((CH,), jnp.int32)])
  starts_ext = jnp.concatenate(
      [netpin_start, jnp.full((Np - N,), P, jnp.int32)])
  ridx = (jnp.arange(NW, dtype=jnp.int32)[:, None] * NB
          + jnp.arange(ROW, dtype=jnp.int32)[None, :])
  starts2d = jnp.where(ridx <= Np, starts_ext[jnp.minimum(ridx, Np)],
                       jnp.int32(P))
  w_ext = jnp.concatenate(
      [net_weights, jnp.zeros((Np - N,), jnp.float32)]).reshape(NW, NB)
  w2d = jnp.pad(w_ext, ((0, 0), (0, NBR - NB)))
  m_ext = jnp.concatenate(
      [net_mask.astype(jnp.float32),
       jnp.zeros((Np - N,), jnp.float32)]).reshape(NW, NB)
  m2d = jnp.pad(m_ext, ((0, 0), (0, NBR - NB)))
  tab_f = sll_counts_table.astype(jnp.float32)

  p0s = starts_ext[jnp.arange(NW, dtype=jnp.int32) * NB]
  p1s = starts_ext[(jnp.arange(NW, dtype=jnp.int32) + 1) * NB]
  a0s = (p0s // 8) * 8
  nchs = (p1s - a0s + CH - 1) // CH
  bounds2d = jnp.stack(
      [p0s, p1s, a0s, nchs] + [jnp.zeros((NW,), jnp.int32)] * 12,
      axis=1)

  steps = []
  s = 1
  while s <= NB:
    steps.append(s)
    s *= 2
  steps = tuple(reversed(steps))

  mesh = plsc.VectorSubcoreMesh(
      core_axis_name="c", subcore_axis_name="s",
      num_cores=NC, num_subcores=NS)

  @functools.partial(
      pl.kernel,
      out_type=jax.ShapeDtypeStruct((NW, L), jnp.float32),
      mesh=mesh,
      compiler_params=pltpu.CompilerParams(needs_layout_passes=False),
      scratch_types=dict(
          s_ref=pltpu.VMEM((ROW,), jnp.int32),
          occ_ref=pltpu.VMEM((NBR,), jnp.int32),
          w_ref=pltpu.VMEM((NBR,), jnp.float32),
          m_ref=pltpu.VMEM((NBR,), jnp.float32),
          idx_ref=pltpu.VMEM((CH,), jnp.int32),
          py_ref=pltpu.VMEM((CH,), jnp.float32),
          b_ref=pltpu.VMEM((16,), jnp.int32),
          tab_ref=pltpu.VMEM((16,), jnp.float32),
          outv_ref=pltpu.VMEM((L,), jnp.float32),
          sem=pltpu.SemaphoreType.DMA,
      ),
  )
  def sll_kernel(posy_hbm, fnp_hbm, starts_hbm, w_hbm, m_hbm, bnd_hbm,
                 tab_hbm, out_hbm, *, s_ref, occ_ref, w_ref, m_ref,
                 idx_ref, py_ref, b_ref, tab_ref, outv_ref, sem):
    wid = lax.axis_index("s") * NC + lax.axis_index("c")
    lanes = lax.iota(jnp.int32, 16)

    pltpu.sync_copy(starts_hbm.at[wid], s_ref)
    pltpu.sync_copy(w_hbm.at[wid], w_ref)
    pltpu.sync_copy(m_hbm.at[wid], m_ref)
    pltpu.sync_copy(bnd_hbm.at[wid], b_ref)
    pltpu.sync_copy(tab_hbm, tab_ref)

    bv = b_ref[...]
    p0 = bv[0]
    p1 = bv[1]
    a0 = bv[2]
    nch = bv[3]

    def zbody(i, _):
      occ_ref[pl.ds(i * 16, 16)] = jnp.zeros((16,), jnp.int32)
      return 0

    lax.fori_loop(0, NBR // 16, zbody, 0)

    ones16 = jnp.full((16,), 1, jnp.int32)
    zeros16 = jnp.zeros((16,), jnp.int32)
    shift_idx = [jnp.maximum(lanes - d, 0) for d in (1, 2, 4, 8)]

    def do_vreg(base, off, car):
      cseg, cbit = car
      p = base + off + lanes
      py = py_ref[pl.ds(off, 16)]
      sy = jnp.clip((py * SLR_INV_H).astype(jnp.int32), 0, NUM_SLRY - 1)
      valid = (p >= p0) & (p < p1)
      bit = jnp.where(valid, jnp.left_shift(ones16, sy), zeros16)
      k = zeros16
      for st in steps:
        cand = k + st
        sv = plsc.load_gather(s_ref, [jnp.minimum(cand, NB)])
        ok = (cand <= NB) & (sv <= p)
        k = jnp.where(ok, cand, k)
      seg = jnp.minimum(k, NB - 1)
      bit = bit | jnp.where(seg == cseg, cbit, 0)
      bacc = bit
      for si in shift_idx:
        sseg = seg.at[si].get(mode="promise_in_bounds")
        sbit = bacc.at[si].get(mode="promise_in_bounds")
        bacc = bacc | jnp.where(sseg == seg, sbit, 0)
      send = plsc.load_gather(s_ref, [seg + 1])
      lastt = p == send - 1
      plsc.store_scatter(occ_ref, [seg], bacc, mask=lastt)
      return (seg[15], bacc[15])

    U = 4

    def chunk_body(c, car):
      base = pl.multiple_of(a0 + c * CH, 8)
      pltpu.sync_copy(fnp_hbm.at[pl.ds(base, CH)], idx_ref)
      pltpu.async_copy(posy_hbm.at[idx_ref], py_ref, sem).wait()

      def vbody(j, car):
        for u in range(U):
          car = do_vreg(base, j * (16 * U) + u * 16, car)
        return car

      return lax.fori_loop(0, CH // (16 * U), vbody, car)

    carry0 = (jnp.int32(-1), jnp.int32(0))
    lax.fori_loop(0, nch, chunk_body, carry0)

    def fbody(i, acc):
      occv = occ_ref[pl.ds(i * 16, 16)]
      sll = plsc.load_gather(tab_ref, [occv])
      w = w_ref[pl.ds(i * 16, 16)]
      m = m_ref[pl.ds(i * 16, 16)]
      return acc + w * m * sll

    acc = lax.fori_loop(0, NBR // 16, fbody,
                        jnp.zeros((L,), jnp.float32))
    outv_ref[...] = acc
    pltpu.sync_copy(outv_ref, out_hbm.at[wid])

  partials = sll_kernel(pos_y, fnp_pad, starts2d, w2d, m2d, bounds2d,
                        tab_f)
  return jnp.sum(partials)


# speedup vs baseline: 653.4783x; 1.7398x over previous
import functools

import jax
import jax.numpy as jnp
from jax import lax
from jax.experimental import pallas as pl
from jax.experimental.pallas import tpu as pltpu
from jax.experimental.pallas import tpu_sc as plsc

SLR_INV_H = 4.0
NUM_SLRY = 4
NC, NS, L = 2, 16, 16
NW = NC * NS
CH = 4096


def _ceil_to(x, m):
  return (x + m - 1) // m * m


@functools.partial(jax.jit, static_argnums=())
def kernel(pos, flat_netpin, netpin_start, net_weights, net_mask,
           sll_counts_table):
  P = flat_netpin.shape[0]
  N = netpin_start.shape[0] - 1
  NB = -(-N // NW)
  ROW = _ceil_to(NB + 1, 16)
  NBR = _ceil_to(NB, 16)
  Np = NW * NB

  pos_y = pos[P:]
  fnp_pad = jnp.concatenate(
      [flat_netpin, jnp.zeros((CH,), jnp.int32)])
  starts_ext = jnp.concatenate(
      [netpin_start, jnp.full((Np - N,), P, jnp.int32)])
  ridx = (jnp.arange(NW, dtype=jnp.int32)[:, None] * NB
          + jnp.arange(ROW, dtype=jnp.int32)[None, :])
  starts2d = jnp.where(ridx <= Np, starts_ext[jnp.minimum(ridx, Np)],
                       jnp.int32(P))
  w_ext = jnp.concatenate(
      [net_weights, jnp.zeros((Np - N,), jnp.float32)]).reshape(NW, NB)
  w2d = jnp.pad(w_ext, ((0, 0), (0, NBR - NB)))
  m_ext = jnp.concatenate(
      [net_mask.astype(jnp.float32),
       jnp.zeros((Np - N,), jnp.float32)]).reshape(NW, NB)
  m2d = jnp.pad(m_ext, ((0, 0), (0, NBR - NB)))
  tab_f = sll_counts_table.astype(jnp.float32)

  p0s = starts_ext[jnp.arange(NW, dtype=jnp.int32) * NB]
  p1s = starts_ext[(jnp.arange(NW, dtype=jnp.int32) + 1) * NB]
  a0s = (p0s // 8) * 8
  nchs = (p1s - a0s + CH - 1) // CH
  bounds2d = jnp.stack(
      [p0s, p1s, a0s, nchs] + [jnp.zeros((NW,), jnp.int32)] * 12,
      axis=1)

  steps = []
  s = 1
  while s <= NB:
    steps.append(s)
    s *= 2
  steps = tuple(reversed(steps))

  mesh = plsc.VectorSubcoreMesh(
      core_axis_name="c", subcore_axis_name="s",
      num_cores=NC, num_subcores=NS)

  @functools.partial(
      pl.kernel,
      out_type=jax.ShapeDtypeStruct((NW, L), jnp.float32),
      mesh=mesh,
      compiler_params=pltpu.CompilerParams(needs_layout_passes=False),
      scratch_types=dict(
          s_ref=pltpu.VMEM((ROW,), jnp.int32),
          occ_ref=pltpu.VMEM((NBR,), jnp.int32),
          w_ref=pltpu.VMEM((NBR,), jnp.float32),
          m_ref=pltpu.VMEM((NBR,), jnp.float32),
          idx_ref=pltpu.VMEM((CH,), jnp.int32),
          py_ref=pltpu.VMEM((CH,), jnp.float32),
          b_ref=pltpu.VMEM((16,), jnp.int32),
          tab_ref=pltpu.VMEM((16,), jnp.float32),
          outv_ref=pltpu.VMEM((L,), jnp.float32),
          sem=pltpu.SemaphoreType.DMA,
      ),
  )
  def sll_kernel(posy_hbm, fnp_hbm, starts_hbm, w_hbm, m_hbm, bnd_hbm,
                 tab_hbm, out_hbm, *, s_ref, occ_ref, w_ref, m_ref,
                 idx_ref, py_ref, b_ref, tab_ref, outv_ref, sem):
    wid = lax.axis_index("s") * NC + lax.axis_index("c")
    lanes = lax.iota(jnp.int32, 16)

    pltpu.sync_copy(starts_hbm.at[wid], s_ref)
    pltpu.sync_copy(w_hbm.at[wid], w_ref)
    pltpu.sync_copy(m_hbm.at[wid], m_ref)
    pltpu.sync_copy(bnd_hbm.at[wid], b_ref)
    pltpu.sync_copy(tab_hbm, tab_ref)

    bv = b_ref[...]
    p0 = bv[0]
    p1 = bv[1]
    a0 = bv[2]
    nch = bv[3]

    def zbody(i, _):
      occ_ref[pl.ds(i * 16, 16)] = jnp.zeros((16,), jnp.int32)
      return 0

    lax.fori_loop(0, NBR // 16, zbody, 0)

    ones16 = jnp.full((16,), 1, jnp.int32)
    zeros16 = jnp.zeros((16,), jnp.int32)
    shift_idx = [jnp.maximum(lanes - d, 0) for d in (1, 2, 4, 8)]

    U = 4

    def chunk_body(c, car):
      base = pl.multiple_of(a0 + c * CH, 8)
      pltpu.sync_copy(fnp_hbm.at[pl.ds(base, CH)], idx_ref)
      pltpu.async_copy(posy_hbm.at[idx_ref], py_ref, sem).wait()

      def vbody(j, car):
        ps, bits, segs = [], [], []
        for u in range(U):
          off = j * (16 * U) + u * 16
          p = base + off + lanes
          py = py_ref[pl.ds(off, 16)]
          sy = jnp.clip((py * SLR_INV_H).astype(jnp.int32), 0,
                        NUM_SLRY - 1)
          valid = (p >= p0) & (p < p1)
          bit = jnp.where(valid, jnp.left_shift(ones16, sy), zeros16)
          k = zeros16
          for st in steps:
            cand = k + st
            sv = plsc.load_gather(s_ref, [jnp.minimum(cand, NB)])
            ok = (cand <= NB) & (sv <= p)
            k = jnp.where(ok, cand, k)
          seg = jnp.minimum(k, NB - 1)
          ps.append(p)
          bits.append(bit)
          segs.append(seg)
        lasts = []
        for u in range(U):
          send = plsc.load_gather(s_ref, [segs[u] + 1])
          lasts.append(ps[u] == send - 1)
        baccs = []
        for u in range(U):
          cseg, cbit = car
          seg = segs[u]
          bacc = bits[u] | jnp.where(seg == cseg, cbit, 0)
          for si in shift_idx:
            sseg = seg.at[si].get(mode="promise_in_bounds")
            sbit = bacc.at[si].get(mode="promise_in_bounds")
            bacc = bacc | jnp.where(sseg == seg, sbit, 0)
          baccs.append(bacc)
          car = (seg[15], bacc[15])
        for u in range(U):
          plsc.store_scatter(occ_ref, [segs[u]], baccs[u], mask=lasts[u])
        return car

      return lax.fori_loop(0, CH // (16 * U), vbody, car)

    carry0 = (jnp.int32(-1), jnp.int32(0))
    lax.fori_loop(0, nch, chunk_body, carry0)

    def fbody(i, acc):
      occv = occ_ref[pl.ds(i * 16, 16)]
      sll = plsc.load_gather(tab_ref, [occv])
      w = w_ref[pl.ds(i * 16, 16)]
      m = m_ref[pl.ds(i * 16, 16)]
      return acc + w * m * sll

    acc = lax.fori_loop(0, NBR // 16, fbody,
                        jnp.zeros((L,), jnp.float32))
    outv_ref[...] = acc
    pltpu.sync_copy(outv_ref, out_hbm.at[wid])

  partials = sll_kernel(pos_y, fnp_pad, starts2d, w2d, m2d, bounds2d,
                        tab_f)
  return jnp.sum(partials)


# speedup vs baseline: 921.8469x; 1.4107x over previous
import functools

import jax
import jax.numpy as jnp
from jax import lax
from jax.experimental import pallas as pl
from jax.experimental.pallas import tpu as pltpu
from jax.experimental.pallas import tpu_sc as plsc

SLR_INV_H = 4.0
NUM_SLRY = 4
NC, NS, L = 2, 16, 16
NW = NC * NS
CH = 4096


def _ceil_to(x, m):
  return (x + m - 1) // m * m


@functools.partial(jax.jit, static_argnums=())
def kernel(pos, flat_netpin, netpin_start, net_weights, net_mask,
           sll_counts_table):
  P = flat_netpin.shape[0]
  N = netpin_start.shape[0] - 1
  NB = -(-N // NW)
  ROW = _ceil_to(NB + 1 + 16, 16)
  NBR = _ceil_to(NB, 16)
  Np = NW * NB

  pos_y = pos[P:]
  fnp_pad = jnp.concatenate(
      [flat_netpin, jnp.zeros((CH,), jnp.int32)])
  ns_ext = jnp.concatenate(
      [netpin_start, jnp.full((Np - N + ROW + 32,), P, jnp.int32)])
  w_ext = jnp.concatenate(
      [net_weights, jnp.zeros((Np - N,), jnp.float32)]).reshape(NW, NB)
  w2d = jnp.pad(w_ext, ((0, 0), (0, NBR - NB)))
  m_ext = jnp.concatenate(
      [net_mask.astype(jnp.float32),
       jnp.zeros((Np - N,), jnp.float32)]).reshape(NW, NB)
  m2d = jnp.pad(m_ext, ((0, 0), (0, NBR - NB)))
  tab_f = sll_counts_table.astype(jnp.float32)

  n0s = jnp.arange(NW, dtype=jnp.int32) * NB
  p0s = lax.slice(netpin_start, (0,), (N,), (NB,))
  p1s = jnp.concatenate(
      [lax.slice(netpin_start, (NB,), (N,), (NB,)), netpin_start[-1:]])
  a0s = (p0s // 8) * 8
  nchs = (p1s - a0s + CH - 1) // CH
  abs_ = (n0s // 8) * 8
  soffs = n0s - abs_
  bounds2d = jnp.stack(
      [p0s, p1s, a0s, nchs, abs_, soffs]
      + [jnp.zeros((NW,), jnp.int32)] * 10,
      axis=1)

  steps = []
  s = 1
  while s <= NB:
    steps.append(s)
    s *= 2
  steps = tuple(reversed(steps))

  mesh = plsc.VectorSubcoreMesh(
      core_axis_name="c", subcore_axis_name="s",
      num_cores=NC, num_subcores=NS)

  @functools.partial(
      pl.kernel,
      out_type=jax.ShapeDtypeStruct((NW, L), jnp.float32),
      mesh=mesh,
      compiler_params=pltpu.CompilerParams(needs_layout_passes=False),
      scratch_types=dict(
          s_raw_ref=pltpu.VMEM((ROW + 16,), jnp.int32),
          s_ref=pltpu.VMEM((ROW,), jnp.int32),
          occ_ref=pltpu.VMEM((NBR,), jnp.int32),
          w_ref=pltpu.VMEM((NBR,), jnp.float32),
          m_ref=pltpu.VMEM((NBR,), jnp.float32),
          idx_ref=pltpu.VMEM((CH,), jnp.int32),
          pya_ref=pltpu.VMEM((CH,), jnp.float32),
          pyb_ref=pltpu.VMEM((CH,), jnp.float32),
          b_ref=pltpu.VMEM((16,), jnp.int32),
          tab_ref=pltpu.VMEM((16,), jnp.float32),
          outv_ref=pltpu.VMEM((L,), jnp.float32),
          sema=pltpu.SemaphoreType.DMA,
          semb=pltpu.SemaphoreType.DMA,
      ),
  )
  def sll_kernel(posy_hbm, fnp_hbm, ns_hbm, w_hbm, m_hbm, bnd_hbm,
                 tab_hbm, out_hbm, *, s_raw_ref, s_ref, occ_ref, w_ref,
                 m_ref, idx_ref, pya_ref, pyb_ref, b_ref, tab_ref,
                 outv_ref, sema, semb):
    wid = lax.axis_index("s") * NC + lax.axis_index("c")
    lanes = lax.iota(jnp.int32, 16)

    pltpu.sync_copy(w_hbm.at[wid], w_ref)
    pltpu.sync_copy(m_hbm.at[wid], m_ref)
    pltpu.sync_copy(bnd_hbm.at[wid], b_ref)
    pltpu.sync_copy(tab_hbm, tab_ref)

    bv = b_ref[...]
    p0 = bv[0]
    p1 = bv[1]
    a0 = bv[2]
    nch = bv[3]
    ab = pl.multiple_of(bv[4], 8)
    s_off = bv[5]

    pltpu.sync_copy(ns_hbm.at[pl.ds(ab, ROW + 16)], s_raw_ref)

    def sbody(i, _):
      s_ref[pl.ds(i * 16, 16)] = s_raw_ref[pl.ds(i * 16 + s_off, 16)]
      return 0

    lax.fori_loop(0, ROW // 16, sbody, 0)

    def zbody(i, _):
      occ_ref[pl.ds(i * 16, 16)] = jnp.zeros((16,), jnp.int32)
      return 0

    lax.fori_loop(0, NBR // 16, zbody, 0)

    ones16 = jnp.full((16,), 1, jnp.int32)
    zeros16 = jnp.zeros((16,), jnp.int32)
    shift_idx = [jnp.maximum(lanes - d, 0) for d in (1, 2, 4, 8)]

    U = 4

    def fire(c, py_ref, sem):
      base = pl.multiple_of(a0 + c * CH, 8)
      pltpu.sync_copy(fnp_hbm.at[pl.ds(base, CH)], idx_ref)
      pltpu.async_copy(posy_hbm.at[idx_ref], py_ref, sem)

    def drain(py_ref, sem):
      pltpu.make_async_copy(posy_hbm.at[pl.ds(0, CH)], py_ref,
                            sem).wait()

    @pl.when(nch > 0)
    def _():
      fire(0, pya_ref, sema)

    def compute_chunk(c, py_ref, car):
      base = pl.multiple_of(a0 + c * CH, 8)

      def vbody(j, car):
        ps, bits, segs = [], [], []
        for u in range(U):
          off = j * (16 * U) + u * 16
          p = base + off + lanes
          py = py_ref[pl.ds(off, 16)]
          sy = jnp.clip((py * SLR_INV_H).astype(jnp.int32), 0,
                        NUM_SLRY - 1)
          valid = (p >= p0) & (p < p1)
          bit = jnp.where(valid, jnp.left_shift(ones16, sy), zeros16)
          k = zeros16
          for st in steps:
            cand = k + st
            sv = plsc.load_gather(s_ref, [jnp.minimum(cand, NB)])
            ok = (cand <= NB) & (sv <= p)
            k = jnp.where(ok, cand, k)
          seg = jnp.minimum(k, NB - 1)
          ps.append(p)
          bits.append(bit)
          segs.append(seg)
        lasts = []
        for u in range(U):
          send = plsc.load_gather(s_ref, [segs[u] + 1])
          lasts.append(ps[u] == send - 1)
        baccs = []
        for u in range(U):
          cseg, cbit = car
          seg = segs[u]
          bacc = bits[u] | jnp.where(seg == cseg, cbit, 0)
          for si in shift_idx:
            sseg = seg.at[si].get(mode="promise_in_bounds")
            sbit = bacc.at[si].get(mode="promise_in_bounds")
            bacc = bacc | jnp.where(sseg == seg, sbit, 0)
          baccs.append(bacc)
          car = (seg[15], bacc[15])
        for u in range(U):
          plsc.store_scatter(occ_ref, [segs[u]], baccs[u], mask=lasts[u])
        return car

      return lax.fori_loop(0, CH // (16 * U), vbody, car)

    def pair_body(i, car):
      c0 = 2 * i
      c1 = c0 + 1
      c2 = c0 + 2
      drain(pya_ref, sema)

      @pl.when(c1 < nch)
      def _():
        fire(c1, pyb_ref, semb)

      car = compute_chunk(c0, pya_ref, car)

      @pl.when(c1 < nch)
      def _():
        drain(pyb_ref, semb)

      @pl.when(c2 < nch)
      def _():
        fire(c2, pya_ref, sema)

      car = compute_chunk(c1, pyb_ref, car)
      return car

    carry0 = (jnp.int32(-1), jnp.int32(0))
    lax.fori_loop(0, (nch + 1) // 2, pair_body, carry0)

    def fbody(i, acc):
      occv = occ_ref[pl.ds(i * 16, 16)]
      sll = plsc.load_gather(tab_ref, [occv])
      w = w_ref[pl.ds(i * 16, 16)]
      m = m_ref[pl.ds(i * 16, 16)]
      return acc + w * m * sll

    acc = lax.fori_loop(0, NBR // 16, fbody,
                        jnp.zeros((L,), jnp.float32))
    outv_ref[...] = acc
    pltpu.sync_copy(outv_ref, out_hbm.at[wid])

  partials = sll_kernel(pos_y, fnp_pad, ns_ext, w2d, m2d, bounds2d,
                        tab_f)
  return jnp.sum(partials)


# speedup vs baseline: 1243.5783x; 1.3490x over previous
import functools

import jax
import jax.numpy as jnp
from jax import lax
from jax.experimental import pallas as pl
from jax.experimental.pallas import tpu as pltpu
from jax.experimental.pallas import tpu_sc as plsc

SLR_INV_H = 4.0
NUM_SLRY = 4
NC, NS, L = 2, 16, 16
NW = NC * NS
CH = 4096


def _ceil_to(x, m):
  return (x + m - 1) // m * m


@functools.partial(jax.jit, static_argnums=())
def kernel(pos, flat_netpin, netpin_start, net_weights, net_mask,
           sll_counts_table):
  P = flat_netpin.shape[0]
  N = netpin_start.shape[0] - 1
  NB = -(-N // NW)
  ROW = _ceil_to(NB + 1 + 16, 16)
  NBR = _ceil_to(NB, 16)
  Np = NW * NB

  pos_y = pos[P:]
  fnp_pad = jnp.concatenate(
      [flat_netpin, jnp.zeros((CH,), jnp.int32)])
  ns_ext = jnp.concatenate(
      [netpin_start, jnp.full((Np - N + ROW + 32,), P, jnp.int32)])
  w_ext = jnp.concatenate(
      [net_weights, jnp.zeros((Np - N,), jnp.float32)]).reshape(NW, NB)
  w2d = jnp.pad(w_ext, ((0, 0), (0, NBR - NB)))
  m_ext = jnp.concatenate(
      [net_mask.astype(jnp.float32),
       jnp.zeros((Np - N,), jnp.float32)]).reshape(NW, NB)
  m2d = jnp.pad(m_ext, ((0, 0), (0, NBR - NB)))
  tab_f = sll_counts_table.astype(jnp.float32)

  n0s = jnp.arange(NW, dtype=jnp.int32) * NB
  p0s = lax.slice(netpin_start, (0,), (N,), (NB,))
  p1s = jnp.concatenate(
      [lax.slice(netpin_start, (NB,), (N,), (NB,)), netpin_start[-1:]])
  a0s = (p0s // 8) * 8
  nchs = (p1s - a0s + CH - 1) // CH
  abs_ = (n0s // 8) * 8
  soffs = n0s - abs_
  bounds2d = jnp.stack(
      [p0s, p1s, a0s, nchs, abs_, soffs]
      + [jnp.zeros((NW,), jnp.int32)] * 10,
      axis=1)

  mesh = plsc.VectorSubcoreMesh(
      core_axis_name="c", subcore_axis_name="s",
      num_cores=NC, num_subcores=NS)

  @functools.partial(
      pl.kernel,
      out_type=jax.ShapeDtypeStruct((NW, L), jnp.float32),
      mesh=mesh,
      compiler_params=pltpu.CompilerParams(needs_layout_passes=False),
      scratch_types=dict(
          s_raw_ref=pltpu.VMEM((ROW + 16,), jnp.int32),
          s_ref=pltpu.VMEM((ROW,), jnp.int32),
          occ_ref=pltpu.VMEM((NBR,), jnp.int32),
          w_ref=pltpu.VMEM((NBR,), jnp.float32),
          m_ref=pltpu.VMEM((NBR,), jnp.float32),
          idx_ref=pltpu.VMEM((CH,), jnp.int32),
          delta_ref=pltpu.VMEM((CH,), jnp.int32),
          pya_ref=pltpu.VMEM((CH,), jnp.float32),
          pyb_ref=pltpu.VMEM((CH,), jnp.float32),
          b_ref=pltpu.VMEM((16,), jnp.int32),
          tab_ref=pltpu.VMEM((16,), jnp.float32),
          outv_ref=pltpu.VMEM((L,), jnp.float32),
          sema=pltpu.SemaphoreType.DMA,
          semb=pltpu.SemaphoreType.DMA,
      ),
  )
  def sll_kernel(posy_hbm, fnp_hbm, ns_hbm, w_hbm, m_hbm, bnd_hbm,
                 tab_hbm, out_hbm, *, s_raw_ref, s_ref, occ_ref, w_ref,
                 m_ref, idx_ref, delta_ref, pya_ref, pyb_ref, b_ref,
                 tab_ref, outv_ref, sema, semb):
    wid = lax.axis_index("s") * NC + lax.axis_index("c")
    lanes = lax.iota(jnp.int32, 16)

    pltpu.sync_copy(w_hbm.at[wid], w_ref)
    pltpu.sync_copy(m_hbm.at[wid], m_ref)
    pltpu.sync_copy(bnd_hbm.at[wid], b_ref)
    pltpu.sync_copy(tab_hbm, tab_ref)

    bv = b_ref[...]
    p0 = bv[0]
    p1 = bv[1]
    a0 = bv[2]
    nch = bv[3]
    ab = pl.multiple_of(bv[4], 8)
    s_off = bv[5]

    pltpu.sync_copy(ns_hbm.at[pl.ds(ab, ROW + 16)], s_raw_ref)

    def sbody(i, _):
      s_ref[pl.ds(i * 16, 16)] = s_raw_ref[pl.ds(i * 16 + s_off, 16)]
      return 0

    lax.fori_loop(0, ROW // 16, sbody, 0)

    def zbody(i, _):
      occ_ref[pl.ds(i * 16, 16)] = jnp.zeros((16,), jnp.int32)
      return 0

    lax.fori_loop(0, NBR // 16, zbody, 0)

    def dzbody(i, _):
      delta_ref[pl.ds(i * 16, 16)] = jnp.zeros((16,), jnp.int32)
      return 0

    lax.fori_loop(0, CH // 16, dzbody, 0)

    ones16 = jnp.full((16,), 1, jnp.int32)
    zeros16 = jnp.zeros((16,), jnp.int32)
    shift_idx = [jnp.maximum(lanes - d, 0) for d in (1, 2, 4, 8)]
    shift_up1 = jnp.minimum(lanes + 1, 15)

    U = 4

    def fire(c, py_ref, sem):
      base = pl.multiple_of(a0 + c * CH, 8)
      pltpu.sync_copy(fnp_hbm.at[pl.ds(base, CH)], idx_ref)
      pltpu.async_copy(posy_hbm.at[idx_ref], py_ref, sem)

    def drain(py_ref, sem):
      pltpu.make_async_copy(posy_hbm.at[pl.ds(0, CH)], py_ref,
                            sem).wait()

    @pl.when(nch > 0)
    def _():
      fire(0, pya_ref, sema)

    def compute_chunk(c, py_ref, car):
      base = pl.multiple_of(a0 + c * CH, 8)
      cseg0, cbit0, cmax0, knet0 = car

      def scond(st):
        return st[1] == 16

      def sbody2(st):
        k, _ = st
        sv = s_ref[pl.ds(k, 16)]
        off = sv - base
        inb = (off >= 0) & (off < CH)
        vals = k + lanes + 1
        nxtoff = off.at[shift_up1].get(mode="promise_in_bounds")
        keep = ((off != nxtoff) | (lanes == 15)) & inb
        plsc.store_scatter(delta_ref, [jnp.clip(off, 0, CH - 1)], vals,
                           mask=keep)
        pc = plsc.all_reduce_population_count(inb)
        cnt = pc[0]
        return (k + cnt, cnt)

      knet1, _ = lax.while_loop(scond, sbody2, (knet0, jnp.int32(16)))
      car = (cseg0, cbit0, cmax0, knet1)

      def vbody(j, car):
        ps, bits, vms = [], [], []
        for u in range(U):
          off = j * (16 * U) + u * 16
          p = base + off + lanes
          py = py_ref[pl.ds(off, 16)]
          sy = jnp.clip((py * SLR_INV_H).astype(jnp.int32), 0,
                        NUM_SLRY - 1)
          valid = (p >= p0) & (p < p1)
          bit = jnp.where(valid, jnp.left_shift(ones16, sy), zeros16)
          dv = delta_ref[pl.ds(off, 16)]
          delta_ref[pl.ds(off, 16)] = zeros16
          vms.append(plsc.cummax(dv))
          ps.append(p)
          bits.append(bit)
        cseg, cbit, cmax, knet = car
        segs = []
        for u in range(U):
          segr = jnp.maximum(vms[u], cmax)
          segs.append(jnp.maximum(segr - 1, 0))
          cmax = segr[15]
        lasts = []
        for u in range(U):
          send = plsc.load_gather(s_ref, [segs[u] + 1])
          lasts.append(ps[u] == send - 1)
        baccs = []
        for u in range(U):
          seg = segs[u]
          bacc = bits[u] | jnp.where(seg == cseg, cbit, 0)
          for si in shift_idx:
            sseg = seg.at[si].get(mode="promise_in_bounds")
            sbit = bacc.at[si].get(mode="promise_in_bounds")
            bacc = bacc | jnp.where(sseg == seg, sbit, 0)
          baccs.append(bacc)
          cseg = seg[15]
          cbit = bacc[15]
        car = (cseg, cbit, cmax, knet)
        for u in range(U):
          plsc.store_scatter(occ_ref, [segs[u]], baccs[u], mask=lasts[u])
        return car

      return lax.fori_loop(0, CH // (16 * U), vbody, car)

    def pair_body(i, car):
      c0 = 2 * i
      c1 = c0 + 1
      c2 = c0 + 2
      drain(pya_ref, sema)

      @pl.when(c1 < nch)
      def _():
        fire(c1, pyb_ref, semb)

      car = compute_chunk(c0, pya_ref, car)

      @pl.when(c1 < nch)
      def _():
        drain(pyb_ref, semb)

      @pl.when(c2 < nch)
      def _():
        fire(c2, pya_ref, sema)

      car = compute_chunk(c1, pyb_ref, car)
      return car

    carry0 = (jnp.int32(-1), jnp.int32(0), jnp.int32(0), jnp.int32(0))
    lax.fori_loop(0, (nch + 1) // 2, pair_body, carry0)

    def fbody(i, acc):
      occv = occ_ref[pl.ds(i * 16, 16)]
      sll = plsc.load_gather(tab_ref, [occv])
      w = w_ref[pl.ds(i * 16, 16)]
      m = m_ref[pl.ds(i * 16, 16)]
      return acc + w * m * sll

    acc = lax.fori_loop(0, NBR // 16, fbody,
                        jnp.zeros((L,), jnp.float32))
    outv_ref[...] = acc
    pltpu.sync_copy(outv_ref, out_hbm.at[wid])

  partials = sll_kernel(pos_y, fnp_pad, ns_ext, w2d, m2d, bounds2d,
                        tab_f)
  return jnp.sum(partials)


# speedup vs baseline: 1316.6983x; 1.0588x over previous
import functools

import jax
import jax.numpy as jnp
from jax import lax
from jax.experimental import pallas as pl
from jax.experimental.pallas import tpu as pltpu
from jax.experimental.pallas import tpu_sc as plsc

SLR_INV_H = 4.0
NUM_SLRY = 4
NC, NS, L = 2, 16, 16
NW = NC * NS
CH = 4096


def _ceil_to(x, m):
  return (x + m - 1) // m * m


@functools.partial(jax.jit, static_argnums=())
def kernel(pos, flat_netpin, netpin_start, net_weights, net_mask,
           sll_counts_table):
  P = flat_netpin.shape[0]
  N = netpin_start.shape[0] - 1
  NB = -(-N // NW)
  ROW = _ceil_to(NB + 1 + 16, 16)
  NBR = _ceil_to(NB, 16)
  Np = NW * NB

  PY = P

  pos_y = pos[P:]
  fnp_pad = jnp.concatenate(
      [flat_netpin, jnp.zeros((CH,), jnp.int32)])
  ns_ext = jnp.concatenate(
      [netpin_start, jnp.full((Np - N + ROW + 32,), P, jnp.int32)])
  w_ext = jnp.concatenate(
      [net_weights, jnp.zeros((Np - N,), jnp.float32)]).reshape(NW, NB)
  w2d = jnp.pad(w_ext, ((0, 0), (0, NBR - NB)))
  m_ext = jnp.concatenate(
      [net_mask.astype(jnp.float32),
       jnp.zeros((Np - N,), jnp.float32)]).reshape(NW, NB)
  m2d = jnp.pad(m_ext, ((0, 0), (0, NBR - NB)))
  tab_f = sll_counts_table.astype(jnp.float32)

  n0s = jnp.arange(NW, dtype=jnp.int32) * NB
  p0s = lax.slice(netpin_start, (0,), (N,), (NB,))
  p1s = jnp.concatenate(
      [lax.slice(netpin_start, (NB,), (N,), (NB,)), netpin_start[-1:]])
  a0s = (p0s // 8) * 8
  nchs = (p1s - a0s + CH - 1) // CH
  abs_ = (n0s // 8) * 8
  soffs = n0s - abs_
  bounds2d = jnp.stack(
      [p0s, p1s, a0s, nchs, abs_, soffs]
      + [jnp.zeros((NW,), jnp.int32)] * 10,
      axis=1)

  mesh = plsc.VectorSubcoreMesh(
      core_axis_name="c", subcore_axis_name="s",
      num_cores=NC, num_subcores=NS)

  @functools.partial(
      pl.kernel,
      out_type=jax.ShapeDtypeStruct((NW, L), jnp.float32),
      mesh=mesh,
      compiler_params=pltpu.CompilerParams(needs_layout_passes=False),
      scratch_types=dict(
          s_raw_ref=pltpu.VMEM((ROW + 16,), jnp.int32),
          s_ref=pltpu.VMEM((ROW,), jnp.int32),
          occ_ref=pltpu.VMEM((NBR,), jnp.int32),
          w_ref=pltpu.VMEM((NBR,), jnp.float32),
          m_ref=pltpu.VMEM((NBR,), jnp.float32),
          idx_ref=pltpu.VMEM((CH,), jnp.int32),
          delta_ref=pltpu.VMEM((CH,), jnp.int32),
          pya_ref=pltpu.VMEM((CH,), jnp.float32),
          pyb_ref=pltpu.VMEM((CH,), jnp.float32),
          b_ref=pltpu.VMEM((16,), jnp.int32),
          tab_ref=pltpu.VMEM((16,), jnp.float32),
          outv_ref=pltpu.VMEM((L,), jnp.float32),
          shared_ref=pltpu.VMEM_SHARED((PY,), jnp.float32),
          sema=pltpu.SemaphoreType.DMA,
          semb=pltpu.SemaphoreType.DMA,
      ),
  )
  def sll_kernel(posy_hbm, fnp_hbm, ns_hbm, w_hbm, m_hbm, bnd_hbm,
                 tab_hbm, out_hbm, *, s_raw_ref, s_ref, occ_ref, w_ref,
                 m_ref, idx_ref, delta_ref, pya_ref, pyb_ref, b_ref,
                 tab_ref, outv_ref, shared_ref, sema, semb):
    wid = lax.axis_index("s") * NC + lax.axis_index("c")
    lanes = lax.iota(jnp.int32, 16)

    pltpu.sync_copy(w_hbm.at[wid], w_ref)
    pltpu.sync_copy(m_hbm.at[wid], m_ref)
    pltpu.sync_copy(bnd_hbm.at[wid], b_ref)
    pltpu.sync_copy(tab_hbm, tab_ref)

    bv = b_ref[...]
    p0 = bv[0]
    p1 = bv[1]
    a0 = bv[2]
    nch = bv[3]
    ab = pl.multiple_of(bv[4], 8)
    s_off = bv[5]

    pltpu.sync_copy(ns_hbm.at[pl.ds(ab, ROW + 16)], s_raw_ref)

    def sbody(i, _):
      s_ref[pl.ds(i * 16, 16)] = s_raw_ref[pl.ds(i * 16 + s_off, 16)]
      return 0

    lax.fori_loop(0, ROW // 16, sbody, 0)

    def zbody(i, _):
      occ_ref[pl.ds(i * 16, 16)] = jnp.zeros((16,), jnp.int32)
      return 0

    lax.fori_loop(0, NBR // 16, zbody, 0)

    def dzbody(i, _):
      delta_ref[pl.ds(i * 16, 16)] = jnp.zeros((16,), jnp.int32)
      return 0

    lax.fori_loop(0, CH // 16, dzbody, 0)

    ones16 = jnp.full((16,), 1, jnp.int32)
    zeros16 = jnp.zeros((16,), jnp.int32)
    shift_idx = [jnp.maximum(lanes - d, 0) for d in (1, 2, 4, 8)]
    shift_up1 = jnp.minimum(lanes + 1, 15)

    U = 4

    @pl.when(lax.axis_index("s") == 0)
    def _():
      pltpu.sync_copy(posy_hbm, shared_ref)

    plsc.subcore_barrier()

    def fire(c, py_ref, sem):
      base = pl.multiple_of(a0 + c * CH, 8)
      pltpu.sync_copy(fnp_hbm.at[pl.ds(base, CH)], idx_ref)
      pltpu.async_copy(shared_ref.at[idx_ref], py_ref, sem)

    def drain(py_ref, sem):
      pltpu.make_async_copy(posy_hbm.at[pl.ds(0, CH)], py_ref,
                            sem).wait()

    @pl.when(nch > 0)
    def _():
      fire(0, pya_ref, sema)

    def compute_chunk(c, py_ref, car):
      base = pl.multiple_of(a0 + c * CH, 8)
      cseg0, cbit0, cmax0, knet0 = car

      def scond(st):
        return st[1] == 16

      def sbody2(st):
        k, _ = st
        sv = s_ref[pl.ds(k, 16)]
        off = sv - base
        inb = (off >= 0) & (off < CH)
        vals = k + lanes + 1
        nxtoff = off.at[shift_up1].get(mode="promise_in_bounds")
        keep = ((off != nxtoff) | (lanes == 15)) & inb
        plsc.store_scatter(delta_ref, [jnp.clip(off, 0, CH - 1)], vals,
                           mask=keep)
        pc = plsc.all_reduce_population_count(inb)
        cnt = pc[0]
        return (k + cnt, cnt)

      knet1, _ = lax.while_loop(scond, sbody2, (knet0, jnp.int32(16)))
      car = (cseg0, cbit0, cmax0, knet1)

      def vbody(j, car):
        ps, bits, vms = [], [], []
        for u in range(U):
          off = j * (16 * U) + u * 16
          p = base + off + lanes
          py = py_ref[pl.ds(off, 16)]
          sy = jnp.clip((py * SLR_INV_H).astype(jnp.int32), 0,
                        NUM_SLRY - 1)
          valid = (p >= p0) & (p < p1)
          bit = jnp.where(valid, jnp.left_shift(ones16, sy), zeros16)
          dv = delta_ref[pl.ds(off, 16)]
          delta_ref[pl.ds(off, 16)] = zeros16
          vms.append(plsc.cummax(dv))
          ps.append(p)
          bits.append(bit)
        cseg, cbit, cmax, knet = car
        segs = []
        for u in range(U):
          segr = jnp.maximum(vms[u], cmax)
          segs.append(jnp.maximum(segr - 1, 0))
          cmax = segr[15]
        lasts = []
        for u in range(U):
          send = plsc.load_gather(s_ref, [segs[u] + 1])
          lasts.append(ps[u] == send - 1)
        baccs = []
        for u in range(U):
          seg = segs[u]
          bacc = bits[u] | jnp.where(seg == cseg, cbit, 0)
          for si in shift_idx:
            sseg = seg.at[si].get(mode="promise_in_bounds")
            sbit = bacc.at[si].get(mode="promise_in_bounds")
            bacc = bacc | jnp.where(sseg == seg, sbit, 0)
          baccs.append(bacc)
          cseg = seg[15]
          cbit = bacc[15]
        car = (cseg, cbit, cmax, knet)
        for u in range(U):
          plsc.store_scatter(occ_ref, [segs[u]], baccs[u], mask=lasts[u])
        return car

      return lax.fori_loop(0, CH // (16 * U), vbody, car)

    def pair_body(i, car):
      c0 = 2 * i
      c1 = c0 + 1
      c2 = c0 + 2
      drain(pya_ref, sema)

      @pl.when(c1 < nch)
      def _():
        fire(c1, pyb_ref, semb)

      car = compute_chunk(c0, pya_ref, car)

      @pl.when(c1 < nch)
      def _():
        drain(pyb_ref, semb)

      @pl.when(c2 < nch)
      def _():
        fire(c2, pya_ref, sema)

      car = compute_chunk(c1, pyb_ref, car)
      return car

    carry0 = (jnp.int32(-1), jnp.int32(0), jnp.int32(0), jnp.int32(0))
    lax.fori_loop(0, (nch + 1) // 2, pair_body, carry0)

    def fbody(i, acc):
      occv = occ_ref[pl.ds(i * 16, 16)]
      sll = plsc.load_gather(tab_ref, [occv])
      w = w_ref[pl.ds(i * 16, 16)]
      m = m_ref[pl.ds(i * 16, 16)]
      return acc + w * m * sll

    acc = lax.fori_loop(0, NBR // 16, fbody,
                        jnp.zeros((L,), jnp.float32))
    outv_ref[...] = acc
    pltpu.sync_copy(outv_ref, out_hbm.at[wid])

  partials = sll_kernel(pos_y, fnp_pad, ns_ext, w2d, m2d, bounds2d,
                        tab_f)
  return jnp.sum(partials)


# speedup vs baseline: 1346.8638x; 1.0229x over previous
import functools

import jax
import jax.numpy as jnp
from jax import lax
from jax.experimental import pallas as pl
from jax.experimental.pallas import tpu as pltpu
from jax.experimental.pallas import tpu_sc as plsc

SLR_INV_H = 4.0
NUM_SLRY = 4
NC, NS, L = 2, 16, 16
NW = NC * NS
CH = 8192


def _ceil_to(x, m):
  return (x + m - 1) // m * m


@functools.partial(jax.jit, static_argnums=())
def kernel(pos, flat_netpin, netpin_start, net_weights, net_mask,
           sll_counts_table):
  P = flat_netpin.shape[0]
  N = netpin_start.shape[0] - 1
  NB = -(-N // NW)
  ROW = _ceil_to(NB + 1 + 16, 16)
  NBR = _ceil_to(NB, 16)
  Np = NW * NB

  PY = P

  pos_y = pos[P:]
  ns_ext = jnp.concatenate(
      [netpin_start, jnp.full((Np - N + ROW + 32,), P, jnp.int32)])
  w_ext = jnp.concatenate(
      [net_weights, jnp.zeros((Np - N,), jnp.float32)]).reshape(NW, NB)
  w2d = jnp.pad(w_ext, ((0, 0), (0, NBR - NB)))
  m_ext = jnp.concatenate(
      [net_mask.astype(jnp.float32),
       jnp.zeros((Np - N,), jnp.float32)]).reshape(NW, NB)
  m2d = jnp.pad(m_ext, ((0, 0), (0, NBR - NB)))
  tab_f = sll_counts_table.astype(jnp.float32)

  n0s = jnp.arange(NW, dtype=jnp.int32) * NB
  p0s = lax.slice(netpin_start, (0,), (N,), (NB,))
  p1s = jnp.concatenate(
      [lax.slice(netpin_start, (NB,), (N,), (NB,)), netpin_start[-1:]])
  a0s = (p0s // 8) * 8
  nchs = (p1s - a0s + CH - 1) // CH
  abs_ = (n0s // 8) * 8
  soffs = n0s - abs_
  bounds2d = jnp.stack(
      [p0s, p1s, a0s, nchs, abs_, soffs]
      + [jnp.zeros((NW,), jnp.int32)] * 10,
      axis=1)

  mesh = plsc.VectorSubcoreMesh(
      core_axis_name="c", subcore_axis_name="s",
      num_cores=NC, num_subcores=NS)

  @functools.partial(
      pl.kernel,
      out_type=jax.ShapeDtypeStruct((NW, L), jnp.float32),
      mesh=mesh,
      compiler_params=pltpu.CompilerParams(needs_layout_passes=False),
      scratch_types=dict(
          s_raw_ref=pltpu.VMEM((ROW + 16,), jnp.int32),
          s_ref=pltpu.VMEM((ROW,), jnp.int32),
          occ_ref=pltpu.VMEM((NBR,), jnp.int32),
          w_ref=pltpu.VMEM((NBR,), jnp.float32),
          m_ref=pltpu.VMEM((NBR,), jnp.float32),
          idx_ref=pltpu.VMEM((CH,), jnp.int32),
          delta_ref=pltpu.VMEM((CH,), jnp.int32),
          pya_ref=pltpu.VMEM((CH,), jnp.float32),
          pyb_ref=pltpu.VMEM((CH,), jnp.float32),
          b_ref=pltpu.VMEM((16,), jnp.int32),
          tab_ref=pltpu.VMEM((16,), jnp.float32),
          outv_ref=pltpu.VMEM((L,), jnp.float32),
          shared_ref=pltpu.VMEM_SHARED((PY,), jnp.float32),
          sema=pltpu.SemaphoreType.DMA,
          semb=pltpu.SemaphoreType.DMA,
      ),
  )
  def sll_kernel(posy_hbm, fnp_hbm, ns_hbm, w_hbm, m_hbm, bnd_hbm,
                 tab_hbm, out_hbm, *, s_raw_ref, s_ref, occ_ref, w_ref,
                 m_ref, idx_ref, delta_ref, pya_ref, pyb_ref, b_ref,
                 tab_ref, outv_ref, shared_ref, sema, semb):
    wid = lax.axis_index("s") * NC + lax.axis_index("c")
    lanes = lax.iota(jnp.int32, 16)

    pltpu.sync_copy(w_hbm.at[wid], w_ref)
    pltpu.sync_copy(m_hbm.at[wid], m_ref)
    pltpu.sync_copy(bnd_hbm.at[wid], b_ref)
    pltpu.sync_copy(tab_hbm, tab_ref)

    bv = b_ref[...]
    p0 = bv[0]
    p1 = bv[1]
    a0 = bv[2]
    nch = bv[3]
    ab = pl.multiple_of(bv[4], 8)
    s_off = bv[5]

    pltpu.sync_copy(ns_hbm.at[pl.ds(ab, ROW + 16)], s_raw_ref)

    def sbody(i, _):
      s_ref[pl.ds(i * 16, 16)] = s_raw_ref[pl.ds(i * 16 + s_off, 16)]
      return 0

    lax.fori_loop(0, ROW // 16, sbody, 0)

    def zbody(i, _):
      occ_ref[pl.ds(i * 16, 16)] = jnp.zeros((16,), jnp.int32)
      return 0

    lax.fori_loop(0, NBR // 16, zbody, 0)

    def dzbody(i, _):
      delta_ref[pl.ds(i * 16, 16)] = jnp.zeros((16,), jnp.int32)
      return 0

    lax.fori_loop(0, CH // 16, dzbody, 0)

    ones16 = jnp.full((16,), 1, jnp.int32)
    zeros16 = jnp.zeros((16,), jnp.int32)
    shift_idx = [jnp.maximum(lanes - d, 0) for d in (1, 2, 4, 8)]
    shift_up1 = jnp.minimum(lanes + 1, 15)

    U = 4

    @pl.when(lax.axis_index("s") == 0)
    def _():
      pltpu.sync_copy(posy_hbm, shared_ref)

    plsc.subcore_barrier()

    def fire(c, py_ref, sem):
      base = pl.multiple_of(
          jnp.minimum(a0 + c * CH, jnp.int32(P - CH)), 8)
      pltpu.sync_copy(fnp_hbm.at[pl.ds(base, CH)], idx_ref)
      pltpu.async_copy(shared_ref.at[idx_ref], py_ref, sem)

    def drain(py_ref, sem):
      pltpu.make_async_copy(posy_hbm.at[pl.ds(0, CH)], py_ref,
                            sem).wait()

    @pl.when(nch > 0)
    def _():
      fire(0, pya_ref, sema)

    def compute_chunk(c, py_ref, car):
      base_l = a0 + c * CH
      base = pl.multiple_of(jnp.minimum(base_l, jnp.int32(P - CH)), 8)
      pmin = jnp.maximum(p0, base_l)
      cseg0, cbit0, cmax0, knet0 = car

      def scond(st):
        return st[1] == 16

      def sbody2(st):
        k, _ = st
        sv = s_ref[pl.ds(k, 16)]
        off = sv - base
        inb = (off >= 0) & (off < CH) & (k + lanes <= NB)
        vals = k + lanes + 1
        nxtoff = off.at[shift_up1].get(mode="promise_in_bounds")
        keep = ((off != nxtoff) | (lanes == 15)) & inb
        plsc.store_scatter(delta_ref, [jnp.clip(off, 0, CH - 1)], vals,
                           mask=keep)
        pc = plsc.all_reduce_population_count(inb)
        cnt = pc[0]
        return (k + cnt, cnt)

      knet1, _ = lax.while_loop(scond, sbody2, (knet0, jnp.int32(16)))
      car = (cseg0, cbit0, cmax0, knet1)

      def vbody(j, car):
        ps, bits, vms = [], [], []
        for u in range(U):
          off = j * (16 * U) + u * 16
          p = base + off + lanes
          py = py_ref[pl.ds(off, 16)]
          sy = jnp.clip((py * SLR_INV_H).astype(jnp.int32), 0,
                        NUM_SLRY - 1)
          valid = (p >= pmin) & (p < p1)
          bit = jnp.where(valid, jnp.left_shift(ones16, sy), zeros16)
          dv = delta_ref[pl.ds(off, 16)]
          delta_ref[pl.ds(off, 16)] = zeros16
          vms.append(plsc.cummax(dv))
          ps.append(p)
          bits.append(bit)
        cseg, cbit, cmax, knet = car
        segs = []
        for u in range(U):
          segr = jnp.maximum(vms[u], cmax)
          segs.append(jnp.maximum(segr - 1, 0))
          cmax = segr[15]
        lasts = []
        for u in range(U):
          send = plsc.load_gather(s_ref, [segs[u] + 1])
          lasts.append(ps[u] == send - 1)
        baccs = []
        for u in range(U):
          seg = segs[u]
          bacc = bits[u] | jnp.where(seg == cseg, cbit, 0)
          for si in shift_idx:
            sseg = seg.at[si].get(mode="promise_in_bounds")
            sbit = bacc.at[si].get(mode="promise_in_bounds")
            bacc = bacc | jnp.where(sseg == seg, sbit, 0)
          baccs.append(bacc)
          cseg = seg[15]
          cbit = bacc[15]
        car = (cseg, cbit, cmax, knet)
        for u in range(U):
          plsc.store_scatter(occ_ref, [segs[u]], baccs[u], mask=lasts[u])
        return car

      return lax.fori_loop(0, CH // (16 * U), vbody, car)

    def pair_body(i, car):
      c0 = 2 * i
      c1 = c0 + 1
      c2 = c0 + 2
      drain(pya_ref, sema)

      @pl.when(c1 < nch)
      def _():
        fire(c1, pyb_ref, semb)

      car = compute_chunk(c0, pya_ref, car)

      @pl.when(c1 < nch)
      def _():
        drain(pyb_ref, semb)

      @pl.when(c2 < nch)
      def _():
        fire(c2, pya_ref, sema)

      car = compute_chunk(c1, pyb_ref, car)
      return car

    carry0 = (jnp.int32(-1), jnp.int32(0), jnp.int32(0), jnp.int32(0))
    lax.fori_loop(0, (nch + 1) // 2, pair_body, carry0)

    def fbody(i, acc):
      occv = occ_ref[pl.ds(i * 16, 16)]
      sll = plsc.load_gather(tab_ref, [occv])
      w = w_ref[pl.ds(i * 16, 16)]
      m = m_ref[pl.ds(i * 16, 16)]
      return acc + w * m * sll

    acc = lax.fori_loop(0, NBR // 16, fbody,
                        jnp.zeros((L,), jnp.float32))
    outv_ref[...] = acc
    pltpu.sync_copy(outv_ref, out_hbm.at[wid])

  partials = sll_kernel(pos_y, flat_netpin, ns_ext, w2d, m2d, bounds2d,
                        tab_f)
  return jnp.sum(partials)


# speedup vs baseline: 1401.4886x; 1.0406x over previous
import functools

import jax
import jax.numpy as jnp
from jax import lax
from jax.experimental import pallas as pl
from jax.experimental.pallas import tpu as pltpu
from jax.experimental.pallas import tpu_sc as plsc

SLR_INV_H = 4.0
NUM_SLRY = 4
NC, NS, L = 2, 16, 16
NW = NC * NS
CH = 8192


def _ceil_to(x, m):
  return (x + m - 1) // m * m


@functools.partial(jax.jit, static_argnums=())
def kernel(pos, flat_netpin, netpin_start, net_weights, net_mask,
           sll_counts_table):
  P = flat_netpin.shape[0]
  N = netpin_start.shape[0] - 1
  NB = -(-N // NW)
  ROW = _ceil_to(NB + 1 + 16, 16)
  NBR = _ceil_to(NB, 16)
  Np = NW * NB

  PY = P
  assert N == NW * NB and NBR <= N and P >= CH

  ns_ext = jnp.concatenate(
      [netpin_start, jnp.full((Np - N + ROW + 32,), P, jnp.int32)])
  wm = net_weights * net_mask.astype(jnp.float32)
  pos2 = pos.reshape(2, P)

  mesh = plsc.VectorSubcoreMesh(
      core_axis_name="c", subcore_axis_name="s",
      num_cores=NC, num_subcores=NS)

  @functools.partial(
      pl.kernel,
      out_type=jax.ShapeDtypeStruct((NW, L), jnp.float32),
      mesh=mesh,
      compiler_params=pltpu.CompilerParams(needs_layout_passes=False),
      scratch_types=dict(
          s_raw_ref=pltpu.VMEM((ROW + 16,), jnp.int32),
          s_ref=pltpu.VMEM((ROW,), jnp.int32),
          occ_ref=pltpu.VMEM((NBR,), jnp.int32),
          w_raw_ref=pltpu.VMEM((ROW + 16,), jnp.float32),
          w_ref=pltpu.VMEM((ROW,), jnp.float32),
          idx_ref=pltpu.VMEM((CH,), jnp.int32),
          delta_ref=pltpu.VMEM((CH,), jnp.int32),
          pya_ref=pltpu.VMEM((CH,), jnp.float32),
          pyb_ref=pltpu.VMEM((CH,), jnp.float32),
          tab_ref=pltpu.VMEM((16,), jnp.int32),
          outv_ref=pltpu.VMEM((L,), jnp.float32),
          shared_ref=pltpu.VMEM_SHARED((PY,), jnp.float32),
          sema=pltpu.SemaphoreType.DMA,
          semb=pltpu.SemaphoreType.DMA,
      ),
  )
  def sll_kernel(pos_hbm, fnp_hbm, ns_hbm, wm_hbm, tab_hbm, out_hbm, *,
                 s_raw_ref, s_ref, occ_ref, w_raw_ref, w_ref, idx_ref,
                 delta_ref, pya_ref, pyb_ref, tab_ref, outv_ref,
                 shared_ref, sema, semb):
    wid = lax.axis_index("s") * NC + lax.axis_index("c")
    lanes = lax.iota(jnp.int32, 16)

    pltpu.sync_copy(tab_hbm, tab_ref)

    n0 = wid * NB
    ab = pl.multiple_of(n0 - lax.rem(n0, 8), 8)
    s_off = n0 - ab

    pltpu.sync_copy(ns_hbm.at[pl.ds(ab, ROW + 16)], s_raw_ref)
    pltpu.sync_copy(wm_hbm.at[pl.ds(ab, NBR)], w_raw_ref.at[pl.ds(0, NBR)])

    def sbody(i, _):
      s_ref[pl.ds(i * 16, 16)] = s_raw_ref[pl.ds(i * 16 + s_off, 16)]
      w_ref[pl.ds(i * 16, 16)] = w_raw_ref[pl.ds(i * 16 + s_off, 16)]
      return 0

    lax.fori_loop(0, ROW // 16, sbody, 0)

    pv = s_ref[pl.ds(0, 16)]
    p0 = pv[0]
    qv = s_ref[pl.ds((NB // 16) * 16, 16)]
    p1 = qv[NB - (NB // 16) * 16]
    a0 = pl.multiple_of(p0 - lax.rem(p0, 8), 8)
    nch = lax.div(p1 - a0 + CH - 1, jnp.int32(CH))

    def zbody(i, _):
      occ_ref[pl.ds(i * 16, 16)] = jnp.zeros((16,), jnp.int32)
      return 0

    lax.fori_loop(0, NBR // 16, zbody, 0)

    def dzbody(i, _):
      delta_ref[pl.ds(i * 16, 16)] = jnp.zeros((16,), jnp.int32)
      return 0

    lax.fori_loop(0, CH // 16, dzbody, 0)

    ones16 = jnp.full((16,), 1, jnp.int32)
    zeros16 = jnp.zeros((16,), jnp.int32)
    shift_idx = [jnp.maximum(lanes - d, 0) for d in (1, 2, 4, 8)]
    shift_up1 = jnp.minimum(lanes + 1, 15)

    U = 4

    @pl.when(lax.axis_index("s") == 0)
    def _():
      pltpu.sync_copy(pos_hbm.at[1], shared_ref)

    plsc.subcore_barrier()

    def fire(c, py_ref, sem):
      base = pl.multiple_of(
          jnp.minimum(a0 + c * CH, jnp.int32(P - CH)), 8)
      pltpu.sync_copy(fnp_hbm.at[pl.ds(base, CH)], idx_ref)
      pltpu.async_copy(shared_ref.at[idx_ref], py_ref, sem)

    def drain(py_ref, sem):
      pltpu.make_async_copy(pos_hbm.at[0, pl.ds(0, CH)], py_ref,
                            sem).wait()

    @pl.when(nch > 0)
    def _():
      fire(0, pya_ref, sema)

    def compute_chunk(c, py_ref, car):
      base_l = a0 + c * CH
      base = pl.multiple_of(jnp.minimum(base_l, jnp.int32(P - CH)), 8)
      pmin = jnp.maximum(p0, base_l)
      cseg0, cbit0, cmax0, knet0 = car

      def scond(st):
        return st[1] == 16

      def sbody2(st):
        k, _ = st
        sv = s_ref[pl.ds(k, 16)]
        off = sv - base
        inb = (off >= 0) & (off < CH) & (k + lanes <= NB)
        vals = k + lanes + 1
        nxtoff = off.at[shift_up1].get(mode="promise_in_bounds")
        keep = ((off != nxtoff) | (lanes == 15)) & inb
        plsc.store_scatter(delta_ref, [jnp.clip(off, 0, CH - 1)], vals,
                           mask=keep)
        pc = plsc.all_reduce_population_count(inb)
        cnt = pc[0]
        return (k + cnt, cnt)

      knet1, _ = lax.while_loop(scond, sbody2, (knet0, jnp.int32(16)))
      car = (cseg0, cbit0, cmax0, knet1)

      def vbody(j, car):
        ps, bits, vms = [], [], []
        for u in range(U):
          off = j * (16 * U) + u * 16
          p = base + off + lanes
          py = py_ref[pl.ds(off, 16)]
          sy = jnp.clip((py * SLR_INV_H).astype(jnp.int32), 0,
                        NUM_SLRY - 1)
          valid = (p >= pmin) & (p < p1)
          bit = jnp.where(valid, jnp.left_shift(ones16, sy), zeros16)
          dv = delta_ref[pl.ds(off, 16)]
          delta_ref[pl.ds(off, 16)] = zeros16
          vms.append(plsc.cummax(dv))
          ps.append(p)
          bits.append(bit)
        cseg, cbit, cmax, knet = car
        segs = []
        for u in range(U):
          segr = jnp.maximum(vms[u], cmax)
          segs.append(jnp.maximum(segr - 1, 0))
          cmax = segr[15]
        lasts = []
        for u in range(U):
          send = plsc.load_gather(s_ref, [segs[u] + 1])
          lasts.append(ps[u] == send - 1)
        baccs = []
        for u in range(U):
          seg = segs[u]
          bacc = bits[u] | jnp.where(seg == cseg, cbit, 0)
          for si in shift_idx:
            sseg = seg.at[si].get(mode="promise_in_bounds")
            sbit = bacc.at[si].get(mode="promise_in_bounds")
            bacc = bacc | jnp.where(sseg == seg, sbit, 0)
          baccs.append(bacc)
          cseg = seg[15]
          cbit = bacc[15]
        car = (cseg, cbit, cmax, knet)
        for u in range(U):
          plsc.store_scatter(occ_ref, [segs[u]], baccs[u], mask=lasts[u])
        return car

      return lax.fori_loop(0, CH // (16 * U), vbody, car)

    def pair_body(i, car):
      c0 = 2 * i
      c1 = c0 + 1
      c2 = c0 + 2
      drain(pya_ref, sema)

      @pl.when(c1 < nch)
      def _():
        fire(c1, pyb_ref, semb)

      car = compute_chunk(c0, pya_ref, car)

      @pl.when(c1 < nch)
      def _():
        drain(pyb_ref, semb)

      @pl.when(c2 < nch)
      def _():
        fire(c2, pya_ref, sema)

      car = compute_chunk(c1, pyb_ref, car)
      return car

    carry0 = (jnp.int32(-1), jnp.int32(0), jnp.int32(0), jnp.int32(0))
    lax.fori_loop(0, (nch + 1) // 2, pair_body, carry0)

    def fbody(i, acc):
      occv = occ_ref[pl.ds(i * 16, 16)]
      sll = plsc.load_gather(tab_ref, [occv]).astype(jnp.float32)
      w = w_ref[pl.ds(i * 16, 16)]
      nmask = (i * 16 + lanes) < NB
      return acc + jnp.where(nmask, w * sll, 0.0)

    acc = lax.fori_loop(0, NBR // 16, fbody,
                        jnp.zeros((L,), jnp.float32))
    outv_ref[...] = acc
    pltpu.sync_copy(outv_ref, out_hbm.at[wid])

  partials = sll_kernel(pos2, flat_netpin, ns_ext, wm, sll_counts_table)
  return jnp.sum(partials)
